# baseline (speedup 1.0000x reference)
import functools

import jax
import jax.numpy as jnp
from jax import lax
from jax.experimental import pallas as pl
from jax.experimental.pallas import tpu as pltpu
from jax.experimental.pallas import tpu_sc as plsc

D_MODEL = 64
SEQ = 200
NUM_CORES = 2
NUM_SUBCORES = 16
NUM_WORKERS = NUM_CORES * NUM_SUBCORES

IDXW = 100
ROWS_PER_CHUNK = 4
CHUNK = ROWS_PER_CHUNK * SEQ


@functools.partial(jax.jit, static_argnames=("batch",))
def _sc_embed(token_ids, token_table, pos_table, *, batch):
    rows_per_worker = batch // NUM_WORKERS
    chunks_per_worker = rows_per_worker // ROWS_PER_CHUNK

    mesh = plsc.VectorSubcoreMesh(
        core_axis_name="c", subcore_axis_name="s",
        num_cores=NUM_CORES, num_subcores=NUM_SUBCORES,
    )

    @functools.partial(
        pl.kernel,
        mesh=mesh,
        compiler_params=pltpu.CompilerParams(use_tc_tiling_on_sc=False),
        out_type=jax.ShapeDtypeStruct((batch, SEQ, D_MODEL), jnp.float32),
        scratch_types=[
            pltpu.VMEM((ROWS_PER_CHUNK, SEQ), jnp.int32),
            pltpu.VMEM((CHUNK, D_MODEL), jnp.float32),
            pltpu.VMEM((SEQ, D_MODEL), jnp.float32),
            pltpu.SemaphoreType.DMA,
        ],
    )
    def body(ids_hbm, table_hbm, pos_hbm, out_hbm, idx_v, rows_v, pos_v, gsem):
        wid = lax.axis_index("s") * NUM_CORES + lax.axis_index("c")
        pltpu.sync_copy(pos_hbm, pos_v)
        base_row = wid * rows_per_worker

        def chunk_body(g, carry):
            brow = base_row + g * ROWS_PER_CHUNK
            pltpu.sync_copy(ids_hbm.at[pl.ds(brow, ROWS_PER_CHUNK)], idx_v)
            copies = [
                pltpu.async_copy(
                    table_hbm.at[idx_v.at[j]],
                    rows_v.at[pl.ds(j * SEQ, SEQ)],
                    gsem,
                )
                for j in range(ROWS_PER_CHUNK)
            ]
            for c in copies:
                c.wait()

            def add_body(r, inner):
                for c in range(D_MODEL // 16):
                    pv = pos_v[r, pl.ds(c * 16, 16)]
                    for rep in range(ROWS_PER_CHUNK):
                        row = rep * SEQ + r
                        rows_v[row, pl.ds(c * 16, 16)] = (
                            rows_v[row, pl.ds(c * 16, 16)] + pv
                        )
                return inner

            lax.fori_loop(0, SEQ, add_body, 0, unroll=False)
            for j in range(ROWS_PER_CHUNK):
                pltpu.sync_copy(
                    rows_v.at[pl.ds(j * SEQ, SEQ)], out_hbm.at[brow + j]
                )
            return carry

        lax.fori_loop(0, chunks_per_worker, chunk_body, 0, unroll=False)

    return body(token_ids, token_table, pos_table)


def kernel(token_ids, token_table, pos_table):
    batch, _ = token_ids.shape
    return _sc_embed(
        token_ids.astype(jnp.int32), token_table, pos_table, batch=batch
    )


# speedup vs baseline: 1.0002x; 1.0002x over previous
import functools

import jax
import jax.numpy as jnp
from jax import lax
from jax.experimental import pallas as pl
from jax.experimental.layout import Format, Layout
from jax.experimental.pallas import tpu as pltpu
from jax.experimental.pallas import tpu_sc as plsc
from jax.sharding import SingleDeviceSharding

D_MODEL = 64
SEQ = 200
NUM_CORES = 2
NUM_SUBCORES = 16
NUM_WORKERS = NUM_CORES * NUM_SUBCORES

IDXW = 100
ROWS_PER_CHUNK = 4
CHUNK = ROWS_PER_CHUNK * SEQ


def _sc_embed(token_ids, token_table, pos_table, *, batch):
    rows_per_worker = batch // NUM_WORKERS
    chunks_per_worker = rows_per_worker // ROWS_PER_CHUNK

    mesh = plsc.VectorSubcoreMesh(
        core_axis_name="c", subcore_axis_name="s",
        num_cores=NUM_CORES, num_subcores=NUM_SUBCORES,
    )

    @functools.partial(
        pl.kernel,
        mesh=mesh,
        compiler_params=pltpu.CompilerParams(use_tc_tiling_on_sc=False),
        out_type=jax.ShapeDtypeStruct((batch, SEQ, D_MODEL), jnp.float32),
        scratch_types=[
            pltpu.VMEM((ROWS_PER_CHUNK, SEQ), jnp.int32),
            pltpu.VMEM((CHUNK, D_MODEL), jnp.float32),
            pltpu.VMEM((SEQ, D_MODEL), jnp.float32),
            pltpu.SemaphoreType.DMA,
        ],
    )
    def body(ids_hbm, table_hbm, pos_hbm, out_hbm, idx_v, rows_v, pos_v, gsem):
        wid = lax.axis_index("s") * NUM_CORES + lax.axis_index("c")
        pltpu.sync_copy(pos_hbm, pos_v)
        base_row = wid * rows_per_worker

        def chunk_body(g, carry):
            brow = base_row + g * ROWS_PER_CHUNK
            pltpu.sync_copy(ids_hbm.at[pl.ds(brow, ROWS_PER_CHUNK)], idx_v)
            copies = [
                pltpu.async_copy(
                    table_hbm.at[idx_v.at[j]],
                    rows_v.at[pl.ds(j * SEQ, SEQ)],
                    gsem,
                )
                for j in range(ROWS_PER_CHUNK)
            ]
            for c in copies:
                c.wait()

            def add_body(r, inner):
                for c in range(D_MODEL // 16):
                    pv = pos_v[r, pl.ds(c * 16, 16)]
                    for rep in range(ROWS_PER_CHUNK):
                        row = rep * SEQ + r
                        rows_v[row, pl.ds(c * 16, 16)] = (
                            rows_v[row, pl.ds(c * 16, 16)] + pv
                        )
                return inner

            lax.fori_loop(0, SEQ, add_body, 0, unroll=False)
            for j in range(ROWS_PER_CHUNK):
                pltpu.sync_copy(
                    rows_v.at[pl.ds(j * SEQ, SEQ)], out_hbm.at[brow + j]
                )
            return carry

        lax.fori_loop(0, chunks_per_worker, chunk_body, 0, unroll=False)

    return body(token_ids, token_table, pos_table)


@functools.cache
def _jitted(batch):
    out_fmt = Format(
        Layout(major_to_minor=(0, 1, 2), tiling=((8,),)),
        SingleDeviceSharding(jax.devices()[0]),
    )
    return jax.jit(
        functools.partial(_sc_embed, batch=batch), out_shardings=out_fmt
    )


def kernel(token_ids, token_table, pos_table):
    batch, _ = token_ids.shape
    return _jitted(batch)(token_ids.astype(jnp.int32), token_table, pos_table)


# speedup vs baseline: 1.4412x; 1.4410x over previous
import functools

import jax
import jax.numpy as jnp
from jax import lax
from jax.experimental import pallas as pl
from jax.experimental.pallas import tpu as pltpu
from jax.experimental.pallas import tpu_sc as plsc

D_MODEL = 64
SEQ = 200
NUM_CORES = 2
NUM_SUBCORES = 16
NUM_WORKERS = NUM_CORES * NUM_SUBCORES

ROWS_PER_CHUNK = 4
CHUNK = ROWS_PER_CHUNK * SEQ


def _sc_embed(ids, table, pos_table, *, batch):
    rows_per_worker = batch // NUM_WORKERS
    n_chunks = rows_per_worker // ROWS_PER_CHUNK
    assert n_chunks % 2 == 0

    mesh = plsc.VectorSubcoreMesh(
        core_axis_name="c", subcore_axis_name="s",
        num_cores=NUM_CORES, num_subcores=NUM_SUBCORES,
    )

    @functools.partial(
        pl.kernel,
        mesh=mesh,
        compiler_params=pltpu.CompilerParams(use_tc_tiling_on_sc=False),
        out_type=jax.ShapeDtypeStruct((batch, SEQ, 128), jnp.float32),
        scratch_types=[
            pltpu.VMEM((ROWS_PER_CHUNK, SEQ), jnp.int32),
            pltpu.VMEM((ROWS_PER_CHUNK, SEQ), jnp.int32),
            pltpu.VMEM((CHUNK, D_MODEL), jnp.float32),
            pltpu.VMEM((CHUNK, D_MODEL), jnp.float32),
            pltpu.VMEM((SEQ, D_MODEL), jnp.float32),
            pltpu.SemaphoreType.DMA,
            pltpu.SemaphoreType.DMA,
            pltpu.SemaphoreType.DMA,
            pltpu.SemaphoreType.DMA,
        ],
    )
    def body(ids_hbm, table_hbm, pos_hbm, out_hbm,
             idx0, idx1, rows0, rows1, pos_v, g0, g1, o0, o1):
        idx = (idx0, idx1)
        rows = (rows0, rows1)
        gsem = (g0, g1)
        osem = (o0, o1)
        wid = lax.axis_index("s") * NUM_CORES + lax.axis_index("c")
        pltpu.sync_copy(pos_hbm, pos_v)
        base_row = wid * rows_per_worker

        def stage_idx(b, g):
            brow = base_row + g * ROWS_PER_CHUNK
            pltpu.sync_copy(ids_hbm.at[pl.ds(brow, ROWS_PER_CHUNK)], idx[b])

        def fire_gathers(b, g):
            for j in range(ROWS_PER_CHUNK):
                pltpu.async_copy(
                    table_hbm.at[idx[b].at[j]],
                    rows[b].at[pl.ds(j * SEQ, SEQ)],
                    gsem[b],
                )

        def drain_gathers(b):
            for j in range(ROWS_PER_CHUNK):
                pltpu.make_async_copy(
                    table_hbm.at[idx[b].at[j]],
                    rows[b].at[pl.ds(j * SEQ, SEQ)],
                    gsem[b],
                ).wait()

        def out_ref(b, g, j):
            brow = base_row + g * ROWS_PER_CHUNK
            return (
                rows[b].at[pl.ds(j * SEQ, SEQ)],
                out_hbm.at[brow + j, :, pl.ds(0, D_MODEL)],
            )

        def fire_store(b, g):
            for j in range(ROWS_PER_CHUNK):
                src, dst = out_ref(b, g, j)
                pltpu.async_copy(src, dst, osem[b])

        def drain_store(b, g):
            for j in range(ROWS_PER_CHUNK):
                src, dst = out_ref(b, g, j)
                pltpu.make_async_copy(src, dst, osem[b]).wait()

        def add_pos(b):
            def add_body(r, inner):
                for c in range(D_MODEL // 16):
                    pv = pos_v[r, pl.ds(c * 16, 16)]
                    for rep in range(ROWS_PER_CHUNK):
                        row = rep * SEQ + r
                        rows[b][row, pl.ds(c * 16, 16)] = (
                            rows[b][row, pl.ds(c * 16, 16)] + pv
                        )
                return inner

            lax.fori_loop(0, SEQ, add_body, 0, unroll=False)

        stage_idx(0, 0)
        fire_gathers(0, 0)
        stage_idx(1, 1)

        def pair_body(gp, carry):
            for b in range(2):
                g = 2 * gp + b
                drain_gathers(b)

                @pl.when(g >= 1)
                def _():
                    drain_store(1 - b, g - 1)

                @pl.when(g + 1 < n_chunks)
                def _():
                    fire_gathers(1 - b, g + 1)

                add_pos(b)
                fire_store(b, g)

                @pl.when(g + 2 < n_chunks)
                def _():
                    stage_idx(b, g + 2)

            return carry

        lax.fori_loop(0, n_chunks // 2, pair_body, 0, unroll=False)
        drain_store((n_chunks - 1) % 2, n_chunks - 1)

    return body(ids, table, pos_table)


def kernel(token_ids, token_table, pos_table):
    batch, _ = token_ids.shape
    o = _sc_embed(
        token_ids.astype(jnp.int32), token_table, pos_table, batch=batch
    )
    return lax.slice(o, (0, 0, 0), (batch, SEQ, D_MODEL))
